# baseline (speedup 1.0000x reference)
import functools

import jax
import jax.numpy as jnp
from jax import lax
from jax.experimental import pallas as pl
from jax.experimental.pallas import tpu as pltpu
from jax.experimental.pallas import tpu_sc as plsc

N = 10000
E = 320000
IN_CH, H1, H2, OUT_CH = 128, 64, 32, 1

NC, NS = 2, 16
NW = NC * NS
C = 128
CH = 80
EW = CH * C
EPAD = NW * EW
NPAD = 10112
RPT = NPAD // NS

_f32 = jnp.float32
FP = 8


def _mesh():
    return plsc.VectorSubcoreMesh(
        core_axis_name="c", subcore_axis_name="s", num_cores=NC, num_subcores=NS
    )


@functools.partial(
    pl.kernel,
    out_type=jax.ShapeDtypeStruct((NC, NPAD, FP), _f32),
    mesh=_mesh(),
    scratch_types=[
        pltpu.VMEM((CH, C), jnp.int32),
        pltpu.VMEM((C, FP), _f32),
        pltpu.VMEM_SHARED((NPAD, FP), _f32),
    ],
    compiler_params=pltpu.CompilerParams(use_tc_tiling_on_sc=False),
    name="gcn_degree",
)
def _deg_kernel(dstw, ones, zrows, out, dst_v, ones_v, acc):
    cid = lax.axis_index("c")
    sid = lax.axis_index("s")
    wid = sid * NC + cid
    r0 = sid * RPT
    pltpu.sync_copy(zrows.at[pl.ds(r0, RPT)], acc.at[pl.ds(r0, RPT)])
    pltpu.sync_copy(ones, ones_v)
    pltpu.sync_copy(dstw.at[wid], dst_v)
    plsc.subcore_barrier()

    def chunk(j, carry):
        pltpu.sync_copy(ones_v, acc.at[dst_v.at[j]], add=True)
        return carry

    lax.fori_loop(0, CH, chunk, 0)
    plsc.subcore_barrier()
    pltpu.sync_copy(acc.at[pl.ds(r0, RPT)], out.at[cid, pl.ds(r0, RPT), :])


NBUF = 3

CH0, CH1 = 132, 36
CHM = max(CH0, CH1)


def _make_agg(F):
    @functools.partial(
        pl.kernel,
        out_type=jax.ShapeDtypeStruct((NC, NPAD, F), _f32),
        mesh=_mesh(),
        scratch_types=[
            pltpu.VMEM((CHM, C), jnp.int32),
            pltpu.VMEM((CHM, C), jnp.int32),
            pltpu.VMEM((2, NBUF, C, F), _f32),
            pltpu.VMEM_SHARED((NPAD, F), _f32),
            pltpu.SemaphoreType.DMA((2, NBUF)),
            pltpu.SemaphoreType.DMA((2, NBUF)),
        ],
        compiler_params=pltpu.CompilerParams(use_tc_tiling_on_sc=False),
        name=f"gcn_agg_f{F}",
    )
    def _agg(hs, srcw, dstw, zrows, out, src_v, dst_v, buf, acc, gsem, ssem):
        cid = lax.axis_index("c")
        sid = lax.axis_index("s")
        wid = (1 - cid) * NS + sid
        ngrp = jnp.where(cid == 1, CH0 // NBUF, CH1 // NBUF)
        r0 = sid * RPT
        pltpu.sync_copy(zrows.at[pl.ds(r0, RPT)], acc.at[pl.ds(r0, RPT)])
        pltpu.sync_copy(srcw.at[wid], src_v)
        pltpu.sync_copy(dstw.at[wid], dst_v)
        plsc.subcore_barrier()

        def start_gather(r, b, j):
            pltpu.async_copy(hs.at[src_v.at[j]], buf.at[r, b], gsem.at[r, b])

        def wait_gather(r, b, j):
            pltpu.make_async_copy(hs.at[src_v.at[j]], buf.at[r, b],
                                  gsem.at[r, b]).wait()

        def start_scatter(r, b, j):
            pltpu.async_copy(buf.at[r, b], acc.at[dst_v.at[j]], ssem.at[r, b],
                             add=True)

        def wait_scatter(r, b, j):
            pltpu.make_async_copy(buf.at[r, b], acc.at[dst_v.at[j]],
                                  ssem.at[r, b]).wait()

        for b in range(NBUF):
            start_gather(0, b, b)
        for b in range(NBUF):
            start_gather(1, b, NBUF + b)

        def pair(gg, carry):
            e0 = (2 * gg) * NBUF
            o0 = e0 + NBUF
            for b in range(NBUF):
                wait_gather(0, b, e0 + b)
                start_scatter(0, b, e0 + b)
            for b in range(NBUF):
                wait_scatter(0, b, e0 + b)
                start_gather(0, b, e0 + 2 * NBUF + b)
            for b in range(NBUF):
                wait_gather(1, b, o0 + b)
                start_scatter(1, b, o0 + b)
            for b in range(NBUF):
                wait_scatter(1, b, o0 + b)
                start_gather(1, b, o0 + 2 * NBUF + b)
            return carry

        lax.fori_loop(0, ngrp // 2 - 1, pair, 0)

        eb = (ngrp - 2) * NBUF
        ob = (ngrp - 1) * NBUF
        for b in range(NBUF):
            wait_gather(0, b, eb + b)
            start_scatter(0, b, eb + b)
        for b in range(NBUF):
            wait_gather(1, b, ob + b)
            start_scatter(1, b, ob + b)
        for b in range(NBUF):
            wait_scatter(0, b, eb + b)
        for b in range(NBUF):
            wait_scatter(1, b, ob + b)

        plsc.subcore_barrier()
        pltpu.sync_copy(acc.at[pl.ds(r0, RPT)], out.at[cid, pl.ds(r0, RPT), :])

    return _agg


_agg64 = _make_agg(H1)
_agg32 = _make_agg(H2)
_agg8 = _make_agg(FP)


BM = 512


def _tc1_body(x_ref, w_ref, cnt_ref, h_ref, hs_ref, dis_ref, dinv_ref):
    deg = cnt_ref[0, :, 0:1] + cnt_ref[1, :, 0:1] + 1.0
    dis = lax.rsqrt(deg)
    dinv = 1.0 / deg
    h = jnp.dot(x_ref[...], w_ref[...], preferred_element_type=_f32)
    h_ref[...] = h
    hs_ref[...] = dis * h
    dis_ref[...] = dis
    dinv_ref[...] = dinv


def _tc_mid_body(agg_ref, h_ref, dis_ref, dinv_ref, b_ref, w_ref, h2_ref, hs2_ref,
                 *, fout, fpad):
    dis = dis_ref[...]
    z = dis * (agg_ref[0] + agg_ref[1]) + dinv_ref[...] * h_ref[...] + b_ref[...]
    a = jnp.maximum(z, 0.0)
    h2 = jnp.dot(a, w_ref[...], preferred_element_type=_f32)
    h2_ref[...] = h2
    hs = dis * h2
    if fpad == fout:
        hs2_ref[...] = hs
    else:
        col = lax.broadcasted_iota(jnp.int32, (BM, fpad), 1)
        hs2_ref[...] = jnp.where(col < fout, hs, 0.0)


def _tc_out_body(agg_ref, h_ref, dis_ref, dinv_ref, b_ref, out_ref):
    out_ref[...] = (
        dis_ref[...] * (agg_ref[0, :, 0:1] + agg_ref[1, :, 0:1])
        + dinv_ref[...] * h_ref[...]
        + b_ref[...]
    )


def _row_spec(f):
    return pl.BlockSpec((BM, f), lambda i: (i, 0))


def _agg_spec(f):
    return pl.BlockSpec((NC, BM, f), lambda i: (0, i, 0))


def _full_spec(shape):
    return pl.BlockSpec(shape, lambda i: tuple(0 for _ in shape))


_GRID = (pl.cdiv(N, BM),)


def _tc1(x, w1, cnt):
    return pl.pallas_call(
        _tc1_body,
        grid=_GRID,
        in_specs=[_row_spec(IN_CH), _full_spec((IN_CH, H1)), _agg_spec(FP)],
        out_specs=[_row_spec(H1), _row_spec(H1), _row_spec(1), _row_spec(1)],
        out_shape=[
            jax.ShapeDtypeStruct((N, H1), _f32),
            jax.ShapeDtypeStruct((N, H1), _f32),
            jax.ShapeDtypeStruct((N, 1), _f32),
            jax.ShapeDtypeStruct((N, 1), _f32),
        ],
    )(x, w1, cnt)


def _tc_mid(agg, h, dis, dinv, b, w, fin, fout, fpad=None):
    fpad = fout if fpad is None else fpad
    return pl.pallas_call(
        functools.partial(_tc_mid_body, fout=fout, fpad=fpad),
        grid=_GRID,
        in_specs=[
            _agg_spec(fin),
            _row_spec(fin),
            _row_spec(1),
            _row_spec(1),
            _full_spec((1, fin)),
            _full_spec((fin, fout)),
        ],
        out_specs=[_row_spec(fout), _row_spec(fpad)],
        out_shape=[
            jax.ShapeDtypeStruct((N, fout), _f32),
            jax.ShapeDtypeStruct((N, fpad), _f32),
        ],
    )(agg, h, dis, dinv, b, w)


def _tc_out(agg, h, dis, dinv, b):
    return pl.pallas_call(
        _tc_out_body,
        grid=_GRID,
        in_specs=[
            _agg_spec(FP),
            _row_spec(1),
            _row_spec(1),
            _row_spec(1),
            _full_spec((1, 1)),
        ],
        out_specs=_row_spec(1),
        out_shape=jax.ShapeDtypeStruct((N, 1), _f32),
    )(agg, h, dis, dinv, b)


def kernel(x, edge_index, W1, b1, W2, b2, W3, b3):
    src = edge_index[0].astype(jnp.int32)
    dst = edge_index[1].astype(jnp.int32)
    pad = EPAD - E
    srcf = jnp.concatenate([src, jnp.zeros((pad,), jnp.int32)])
    dstf = jnp.concatenate([dst, jnp.full((pad,), N, jnp.int32)])
    dstp = dstf.reshape(NW, CH, C)

    def _split(a, fill):
        cap0 = NS * CH0 * C
        cap1 = NS * CH1 * C
        a0 = a[:cap0].reshape(NS, CH0, C)
        a1 = jnp.concatenate(
            [a[cap0:E], jnp.full((cap1 - (E - cap0),), fill, jnp.int32)]
        ).reshape(NS, CH1, C)
        pad1 = jnp.full((NS, CHM - CH1, C), fill, jnp.int32)
        return jnp.concatenate([a0, jnp.concatenate([a1, pad1], axis=1)], axis=0)

    srcp = _split(srcf, 0)
    dstp_a = _split(dstf, N)

    z64 = jnp.zeros((NPAD, H1), _f32)
    z32 = jnp.zeros((NPAD, H2), _f32)
    z8 = jnp.zeros((NPAD, FP), _f32)
    ones = jnp.ones((C, FP), _f32)

    cnt = _deg_kernel(dstp, ones, z8)
    h1, hs1, dis, dinv = _tc1(x, W1, cnt)
    agg1 = _agg64(hs1, srcp, dstp_a, z64)
    h2, hs2 = _tc_mid(agg1, h1, dis, dinv, b1.reshape(1, H1), W2, H1, H2)
    agg2 = _agg32(hs2, srcp, dstp_a, z32)
    h3, hs3 = _tc_mid(agg2, h2, dis, dinv, b2.reshape(1, H2), W3, H2, OUT_CH, FP)
    agg3 = _agg8(hs3, srcp, dstp_a, z8)
    return _tc_out(agg3, h3, dis, dinv, b3.reshape(1, 1))


# speedup vs baseline: 1.6540x; 1.6540x over previous
import functools

import jax
import jax.numpy as jnp
from jax import lax
from jax.experimental import pallas as pl
from jax.experimental.pallas import tpu as pltpu
from jax.experimental.pallas import tpu_sc as plsc

N = 10000
E = 320000
IN_CH, H1, H2, OUT_CH = 128, 64, 32, 1

NC, NS = 2, 16
NW = NC * NS
C = 128
CH = 80
EW = CH * C
EPAD = NW * EW
NPAD = 10112
RPT = NPAD // NS

_f32 = jnp.float32
FP = 8


def _mesh():
    return plsc.VectorSubcoreMesh(
        core_axis_name="c", subcore_axis_name="s", num_cores=NC, num_subcores=NS
    )


@functools.partial(
    pl.kernel,
    out_type=jax.ShapeDtypeStruct((NC, NPAD, FP), _f32),
    mesh=_mesh(),
    scratch_types=[
        pltpu.VMEM((CH, C), jnp.int32),
        pltpu.VMEM((C, FP), _f32),
        pltpu.VMEM_SHARED((NPAD, FP), _f32),
    ],
    compiler_params=pltpu.CompilerParams(use_tc_tiling_on_sc=False),
    name="gcn_degree",
)
def _deg_kernel(dstw, ones, zrows, out, dst_v, ones_v, acc):
    cid = lax.axis_index("c")
    sid = lax.axis_index("s")
    wid = sid * NC + cid
    r0 = sid * RPT
    pltpu.sync_copy(zrows.at[pl.ds(r0, RPT)], acc.at[pl.ds(r0, RPT)])
    pltpu.sync_copy(ones, ones_v)
    pltpu.sync_copy(dstw.at[wid], dst_v)
    plsc.subcore_barrier()

    def chunk(j, carry):
        pltpu.sync_copy(ones_v, acc.at[dst_v.at[j]], add=True)
        return carry

    lax.fori_loop(0, CH, chunk, 0)
    plsc.subcore_barrier()
    pltpu.sync_copy(acc.at[pl.ds(r0, RPT)], out.at[cid, pl.ds(r0, RPT), :])


NBUF = 4
NGRP = CH // NBUF


def _make_agg(F):
    @functools.partial(
        pl.kernel,
        out_type=jax.ShapeDtypeStruct((NC, NPAD, F), _f32),
        mesh=_mesh(),
        scratch_types=[
            pltpu.VMEM((CH, C), jnp.int32),
            pltpu.VMEM((CH, C), jnp.int32),
            pltpu.VMEM((2, NBUF, C, F), _f32),
            pltpu.VMEM_SHARED((NPAD, F), _f32),
            pltpu.SemaphoreType.DMA((2, NBUF)),
            pltpu.SemaphoreType.DMA((2, NBUF)),
        ],
        compiler_params=pltpu.CompilerParams(use_tc_tiling_on_sc=False),
        name=f"gcn_agg_f{F}",
    )
    def _agg(hs, srcw, dstw, zrows, out, src_v, dst_v, buf, acc, gsem, ssem):
        cid = lax.axis_index("c")
        sid = lax.axis_index("s")
        wid = cid * NS + sid
        r0 = sid * RPT
        pltpu.sync_copy(zrows.at[pl.ds(r0, RPT)], acc.at[pl.ds(r0, RPT)])
        pltpu.sync_copy(srcw.at[wid], src_v)
        pltpu.sync_copy(dstw.at[wid], dst_v)
        plsc.subcore_barrier()

        def start_gather(r, b, j):
            pltpu.async_copy(hs.at[src_v.at[j]], buf.at[r, b], gsem.at[r, b])

        def wait_gather(r, b, j):
            pltpu.make_async_copy(hs.at[src_v.at[j]], buf.at[r, b],
                                  gsem.at[r, b]).wait()

        def start_scatter(r, b, j):
            pltpu.async_copy(buf.at[r, b], acc.at[dst_v.at[j]], ssem.at[r, b],
                             add=True)

        def wait_scatter(r, b, j):
            pltpu.make_async_copy(buf.at[r, b], acc.at[dst_v.at[j]],
                                  ssem.at[r, b]).wait()

        for b in range(NBUF):
            start_gather(0, b, b)
        for b in range(NBUF):
            start_gather(1, b, NBUF + b)

        def pair(gg, carry):
            e0 = (2 * gg) * NBUF
            o0 = e0 + NBUF
            for b in range(NBUF):
                wait_gather(0, b, e0 + b)
                start_scatter(0, b, e0 + b)
            for b in range(NBUF):
                wait_scatter(0, b, e0 + b)
                start_gather(0, b, e0 + 2 * NBUF + b)
            for b in range(NBUF):
                wait_gather(1, b, o0 + b)
                start_scatter(1, b, o0 + b)
            for b in range(NBUF):
                wait_scatter(1, b, o0 + b)
                start_gather(1, b, o0 + 2 * NBUF + b)
            return carry

        lax.fori_loop(0, NGRP // 2 - 1, pair, 0)

        eb = (NGRP - 2) * NBUF
        ob = (NGRP - 1) * NBUF
        for b in range(NBUF):
            wait_gather(0, b, eb + b)
            start_scatter(0, b, eb + b)
        for b in range(NBUF):
            wait_gather(1, b, ob + b)
            start_scatter(1, b, ob + b)
        for b in range(NBUF):
            wait_scatter(0, b, eb + b)
        for b in range(NBUF):
            wait_scatter(1, b, ob + b)

        plsc.subcore_barrier()
        pltpu.sync_copy(acc.at[pl.ds(r0, RPT)], out.at[cid, pl.ds(r0, RPT), :])

    return _agg


_agg64 = _make_agg(H1)
_agg32 = _make_agg(H2)
_agg8 = _make_agg(FP)


BM = 512


def _tc1_body(x_ref, w_ref, cnt_ref, h_ref, hs_ref, dis_ref, dinv_ref):
    deg = cnt_ref[0, :, 0:1] + cnt_ref[1, :, 0:1] + 1.0
    dis = lax.rsqrt(deg)
    dinv = 1.0 / deg
    h = jnp.dot(x_ref[...], w_ref[...], preferred_element_type=_f32)
    h_ref[...] = h
    hs_ref[...] = dis * h
    dis_ref[...] = dis
    dinv_ref[...] = dinv


def _tc_mid_body(agg_ref, h_ref, dis_ref, dinv_ref, b_ref, w_ref, h2_ref, hs2_ref,
                 *, fout, fpad):
    dis = dis_ref[...]
    z = dis * (agg_ref[0] + agg_ref[1]) + dinv_ref[...] * h_ref[...] + b_ref[...]
    a = jnp.maximum(z, 0.0)
    h2 = jnp.dot(a, w_ref[...], preferred_element_type=_f32)
    h2_ref[...] = h2
    hs = dis * h2
    if fpad == fout:
        hs2_ref[...] = hs
    else:
        col = lax.broadcasted_iota(jnp.int32, (BM, fpad), 1)
        hs2_ref[...] = jnp.where(col < fout, hs, 0.0)


def _tc_out_body(agg_ref, h_ref, dis_ref, dinv_ref, b_ref, out_ref):
    out_ref[...] = (
        dis_ref[...] * (agg_ref[0, :, 0:1] + agg_ref[1, :, 0:1])
        + dinv_ref[...] * h_ref[...]
        + b_ref[...]
    )


def _row_spec(f):
    return pl.BlockSpec((BM, f), lambda i: (i, 0))


def _agg_spec(f):
    return pl.BlockSpec((NC, BM, f), lambda i: (0, i, 0))


def _full_spec(shape):
    return pl.BlockSpec(shape, lambda i: tuple(0 for _ in shape))


_GRID = (pl.cdiv(N, BM),)


def _tc1(x, w1, cnt):
    return pl.pallas_call(
        _tc1_body,
        grid=_GRID,
        in_specs=[_row_spec(IN_CH), _full_spec((IN_CH, H1)), _agg_spec(FP)],
        out_specs=[_row_spec(H1), _row_spec(H1), _row_spec(1), _row_spec(1)],
        out_shape=[
            jax.ShapeDtypeStruct((N, H1), _f32),
            jax.ShapeDtypeStruct((N, H1), _f32),
            jax.ShapeDtypeStruct((N, 1), _f32),
            jax.ShapeDtypeStruct((N, 1), _f32),
        ],
    )(x, w1, cnt)


def _tc_mid(agg, h, dis, dinv, b, w, fin, fout, fpad=None):
    fpad = fout if fpad is None else fpad
    return pl.pallas_call(
        functools.partial(_tc_mid_body, fout=fout, fpad=fpad),
        grid=_GRID,
        in_specs=[
            _agg_spec(fin),
            _row_spec(fin),
            _row_spec(1),
            _row_spec(1),
            _full_spec((1, fin)),
            _full_spec((fin, fout)),
        ],
        out_specs=[_row_spec(fout), _row_spec(fpad)],
        out_shape=[
            jax.ShapeDtypeStruct((N, fout), _f32),
            jax.ShapeDtypeStruct((N, fpad), _f32),
        ],
    )(agg, h, dis, dinv, b, w)


def _tc_out(agg, h, dis, dinv, b):
    return pl.pallas_call(
        _tc_out_body,
        grid=_GRID,
        in_specs=[
            _agg_spec(FP),
            _row_spec(1),
            _row_spec(1),
            _row_spec(1),
            _full_spec((1, 1)),
        ],
        out_specs=_row_spec(1),
        out_shape=jax.ShapeDtypeStruct((N, 1), _f32),
    )(agg, h, dis, dinv, b)


def kernel(x, edge_index, W1, b1, W2, b2, W3, b3):
    src = edge_index[0].astype(jnp.int32)
    dst = edge_index[1].astype(jnp.int32)
    pad = EPAD - E
    pad_dst = N + (jnp.arange(pad, dtype=jnp.int32) % (NPAD - N))
    srcp = jnp.concatenate([src, jnp.zeros((pad,), jnp.int32)]).reshape(NW, CH, C)
    dstp = jnp.concatenate([dst, pad_dst]).reshape(NW, CH, C)

    z64 = jnp.zeros((NPAD, H1), _f32)
    z32 = jnp.zeros((NPAD, H2), _f32)
    z8 = jnp.zeros((NPAD, FP), _f32)
    ones = jnp.ones((C, FP), _f32)

    cnt = _deg_kernel(dstp, ones, z8)
    h1, hs1, dis, dinv = _tc1(x, W1, cnt)
    agg1 = _agg64(hs1, srcp, dstp, z64)
    h2, hs2 = _tc_mid(agg1, h1, dis, dinv, b1.reshape(1, H1), W2, H1, H2)
    agg2 = _agg32(hs2, srcp, dstp, z32)
    h3, hs3 = _tc_mid(agg2, h2, dis, dinv, b2.reshape(1, H2), W3, H2, OUT_CH, FP)
    agg3 = _agg8(hs3, srcp, dstp, z8)
    return _tc_out(agg3, h3, dis, dinv, b3.reshape(1, 1))


# speedup vs baseline: 1.8161x; 1.0980x over previous
import functools

import jax
import jax.numpy as jnp
from jax import lax
from jax.experimental import pallas as pl
from jax.experimental.pallas import tpu as pltpu
from jax.experimental.pallas import tpu_sc as plsc

N = 10000
E = 320000
IN_CH, H1, H2, OUT_CH = 128, 64, 32, 1

NC, NS = 2, 16
NW = NC * NS
C = 128
CH = 80
EW = CH * C
EPAD = NW * EW
NPAD = 10112
RPT = NPAD // NS

_f32 = jnp.float32
FP = 8


def _mesh():
    return plsc.VectorSubcoreMesh(
        core_axis_name="c", subcore_axis_name="s", num_cores=NC, num_subcores=NS
    )


@functools.partial(
    pl.kernel,
    out_type=jax.ShapeDtypeStruct((NC, NPAD, FP), _f32),
    mesh=_mesh(),
    scratch_types=[
        pltpu.VMEM((CH, C), jnp.int32),
        pltpu.VMEM((C, FP), _f32),
        pltpu.VMEM_SHARED((NPAD, FP), _f32),
    ],
    compiler_params=pltpu.CompilerParams(use_tc_tiling_on_sc=False),
    name="gcn_degree",
)
def _deg_kernel(dstw, ones, zrows, out, dst_v, ones_v, acc):
    cid = lax.axis_index("c")
    sid = lax.axis_index("s")
    wid = sid * NC + cid
    r0 = sid * RPT
    pltpu.sync_copy(zrows.at[pl.ds(r0, RPT)], acc.at[pl.ds(r0, RPT)])
    pltpu.sync_copy(ones, ones_v)
    pltpu.sync_copy(dstw.at[wid], dst_v)
    plsc.subcore_barrier()

    def chunk(j, carry):
        pltpu.sync_copy(ones_v, acc.at[dst_v.at[j]], add=True)
        return carry

    lax.fori_loop(0, CH, chunk, 0)
    plsc.subcore_barrier()
    pltpu.sync_copy(acc.at[pl.ds(r0, RPT)], out.at[cid, pl.ds(r0, RPT), :])


NBUF = 4

_SPLIT = {64: (136, 24), 32: (128, 32), 8: (120, 40)}


def _make_agg(F):
    ch0, ch1 = _SPLIT[F]
    chm = max(ch0, ch1)
    stream_dst = F >= 64
    dst_scr = (pltpu.VMEM((2, NBUF, C), jnp.int32) if stream_dst
               else pltpu.VMEM((chm, C), jnp.int32))

    @functools.partial(
        pl.kernel,
        out_type=jax.ShapeDtypeStruct((NC, NPAD, F), _f32),
        mesh=_mesh(),
        scratch_types=[
            pltpu.VMEM((chm, C), jnp.int32),
            dst_scr,
            pltpu.VMEM((2, NBUF, C, F), _f32),
            pltpu.VMEM_SHARED((NPAD, F), _f32),
            pltpu.SemaphoreType.DMA((2, NBUF)),
            pltpu.SemaphoreType.DMA((2, NBUF)),
            pltpu.SemaphoreType.DMA((2, NBUF)),
        ],
        compiler_params=pltpu.CompilerParams(use_tc_tiling_on_sc=False),
        name=f"gcn_agg_f{F}",
    )
    def _agg(hs, srcw, dstw, zrows, out, src_v, dst_v, buf, acc, gsem, ssem,
             dsem):
        cid = lax.axis_index("c")
        sid = lax.axis_index("s")
        wid = cid * NS + sid
        ngrp = jnp.where(cid == 0, ch0 // NBUF, ch1 // NBUF)
        r0 = sid * RPT
        pltpu.sync_copy(zrows.at[pl.ds(r0, RPT)], acc.at[pl.ds(r0, RPT)])
        pltpu.sync_copy(srcw.at[wid], src_v)
        if not stream_dst:
            pltpu.sync_copy(dstw.at[wid], dst_v)
        plsc.subcore_barrier()

        def start_gather(r, b, j):
            pltpu.async_copy(hs.at[src_v.at[j]], buf.at[r, b], gsem.at[r, b])
            if stream_dst:
                pltpu.async_copy(dstw.at[wid, j], dst_v.at[r, b],
                                 dsem.at[r, b])

        def wait_gather(r, b, j):
            pltpu.make_async_copy(hs.at[src_v.at[j]], buf.at[r, b],
                                  gsem.at[r, b]).wait()
            if stream_dst:
                pltpu.make_async_copy(dstw.at[wid, j], dst_v.at[r, b],
                                      dsem.at[r, b]).wait()

        def _dst_idx(r, b, j):
            return dst_v.at[r, b] if stream_dst else dst_v.at[j]

        def start_scatter(r, b, j):
            pltpu.async_copy(buf.at[r, b], acc.at[_dst_idx(r, b, j)],
                             ssem.at[r, b], add=True)

        def wait_scatter(r, b, j):
            pltpu.make_async_copy(buf.at[r, b], acc.at[_dst_idx(r, b, j)],
                                  ssem.at[r, b]).wait()

        for b in range(NBUF):
            start_gather(0, b, b)
        for b in range(NBUF):
            start_gather(1, b, NBUF + b)

        def pair(gg, carry):
            e0 = (2 * gg) * NBUF
            o0 = e0 + NBUF
            for b in range(NBUF):
                wait_gather(0, b, e0 + b)
                start_scatter(0, b, e0 + b)
            for b in range(NBUF):
                wait_scatter(0, b, e0 + b)
                start_gather(0, b, e0 + 2 * NBUF + b)
            for b in range(NBUF):
                wait_gather(1, b, o0 + b)
                start_scatter(1, b, o0 + b)
            for b in range(NBUF):
                wait_scatter(1, b, o0 + b)
                start_gather(1, b, o0 + 2 * NBUF + b)
            return carry

        lax.fori_loop(0, ngrp // 2 - 1, pair, 0)

        eb = (ngrp - 2) * NBUF
        ob = (ngrp - 1) * NBUF
        for b in range(NBUF):
            wait_gather(0, b, eb + b)
            start_scatter(0, b, eb + b)
        for b in range(NBUF):
            wait_gather(1, b, ob + b)
            start_scatter(1, b, ob + b)
        for b in range(NBUF):
            wait_scatter(0, b, eb + b)
        for b in range(NBUF):
            wait_scatter(1, b, ob + b)

        plsc.subcore_barrier()
        pltpu.sync_copy(acc.at[pl.ds(r0, RPT)], out.at[cid, pl.ds(r0, RPT), :])

    return _agg


_agg64 = _make_agg(H1)
_agg32 = _make_agg(H2)
_agg8 = _make_agg(FP)


BM = 512


def _tc1_body(x_ref, w_ref, cnt_ref, h_ref, hs_ref, dis_ref, dinv_ref):
    deg = cnt_ref[0, :, 0:1] + cnt_ref[1, :, 0:1] + 1.0
    dis = lax.rsqrt(deg)
    dinv = 1.0 / deg
    h = jnp.dot(x_ref[...], w_ref[...], preferred_element_type=_f32)
    h_ref[...] = h
    hs_ref[...] = dis * h
    dis_ref[...] = dis
    dinv_ref[...] = dinv


def _tc_mid_body(agg_ref, h_ref, dis_ref, dinv_ref, b_ref, w_ref, h2_ref, hs2_ref,
                 *, fout, fpad):
    dis = dis_ref[...]
    z = dis * (agg_ref[0] + agg_ref[1]) + dinv_ref[...] * h_ref[...] + b_ref[...]
    a = jnp.maximum(z, 0.0)
    h2 = jnp.dot(a, w_ref[...], preferred_element_type=_f32)
    h2_ref[...] = h2
    hs = dis * h2
    if fpad == fout:
        hs2_ref[...] = hs
    else:
        col = lax.broadcasted_iota(jnp.int32, (BM, fpad), 1)
        hs2_ref[...] = jnp.where(col < fout, hs, 0.0)


def _tc_out_body(agg_ref, h_ref, dis_ref, dinv_ref, b_ref, out_ref):
    out_ref[...] = (
        dis_ref[...] * (agg_ref[0, :, 0:1] + agg_ref[1, :, 0:1])
        + dinv_ref[...] * h_ref[...]
        + b_ref[...]
    )


def _row_spec(f):
    return pl.BlockSpec((BM, f), lambda i: (i, 0))


def _agg_spec(f):
    return pl.BlockSpec((NC, BM, f), lambda i: (0, i, 0))


def _full_spec(shape):
    return pl.BlockSpec(shape, lambda i: tuple(0 for _ in shape))


_GRID = (pl.cdiv(N, BM),)


def _tc1(x, w1, cnt):
    return pl.pallas_call(
        _tc1_body,
        grid=_GRID,
        in_specs=[_row_spec(IN_CH), _full_spec((IN_CH, H1)), _agg_spec(FP)],
        out_specs=[_row_spec(H1), _row_spec(H1), _row_spec(1), _row_spec(1)],
        out_shape=[
            jax.ShapeDtypeStruct((N, H1), _f32),
            jax.ShapeDtypeStruct((N, H1), _f32),
            jax.ShapeDtypeStruct((N, 1), _f32),
            jax.ShapeDtypeStruct((N, 1), _f32),
        ],
    )(x, w1, cnt)


def _tc_mid(agg, h, dis, dinv, b, w, fin, fout, fpad=None):
    fpad = fout if fpad is None else fpad
    return pl.pallas_call(
        functools.partial(_tc_mid_body, fout=fout, fpad=fpad),
        grid=_GRID,
        in_specs=[
            _agg_spec(fin),
            _row_spec(fin),
            _row_spec(1),
            _row_spec(1),
            _full_spec((1, fin)),
            _full_spec((fin, fout)),
        ],
        out_specs=[_row_spec(fout), _row_spec(fpad)],
        out_shape=[
            jax.ShapeDtypeStruct((N, fout), _f32),
            jax.ShapeDtypeStruct((N, fpad), _f32),
        ],
    )(agg, h, dis, dinv, b, w)


def _tc_out(agg, h, dis, dinv, b):
    return pl.pallas_call(
        _tc_out_body,
        grid=_GRID,
        in_specs=[
            _agg_spec(FP),
            _row_spec(1),
            _row_spec(1),
            _row_spec(1),
            _full_spec((1, 1)),
        ],
        out_specs=_row_spec(1),
        out_shape=jax.ShapeDtypeStruct((N, 1), _f32),
    )(agg, h, dis, dinv, b)


def kernel(x, edge_index, W1, b1, W2, b2, W3, b3):
    src = edge_index[0].astype(jnp.int32)
    dst = edge_index[1].astype(jnp.int32)
    pad = EPAD - E
    pad_dst = N + (jnp.arange(pad, dtype=jnp.int32) % (NPAD - N))
    srcf = jnp.concatenate([src, jnp.zeros((pad,), jnp.int32)])
    dstf = jnp.concatenate([dst, pad_dst])
    dstp = dstf.reshape(NW, CH, C)

    def _asym(a, fill, ch0, ch1):
        chm = max(ch0, ch1)
        cap0 = NS * ch0 * C
        a0 = a[:cap0].reshape(NS, ch0, C)
        a1 = a[cap0:].reshape(NS, ch1, C)
        pad1 = jnp.full((NS, chm - ch1, C), fill, jnp.int32)
        return jnp.concatenate([a0, jnp.concatenate([a1, pad1], axis=1)], axis=0)

    layouts = {
        f: (_asym(srcf, 0, c0, c1), _asym(dstf, N, c0, c1))
        for f, (c0, c1) in _SPLIT.items()
    }

    z64 = jnp.zeros((NPAD, H1), _f32)
    z32 = jnp.zeros((NPAD, H2), _f32)
    z8 = jnp.zeros((NPAD, FP), _f32)
    ones = jnp.ones((C, FP), _f32)

    cnt = _deg_kernel(dstp, ones, z8)
    h1, hs1, dis, dinv = _tc1(x, W1, cnt)
    agg1 = _agg64(hs1, *layouts[H1], z64)
    h2, hs2 = _tc_mid(agg1, h1, dis, dinv, b1.reshape(1, H1), W2, H1, H2)
    agg2 = _agg32(hs2, *layouts[H2], z32)
    h3, hs3 = _tc_mid(agg2, h2, dis, dinv, b2.reshape(1, H2), W3, H2, OUT_CH, FP)
    agg3 = _agg8(hs3, *layouts[FP], z8)
    return _tc_out(agg3, h3, dis, dinv, b3.reshape(1, 1))


# speedup vs baseline: 3.7093x; 2.0424x over previous
import functools

import jax
import jax.numpy as jnp
from jax import lax
from jax.experimental import pallas as pl
from jax.experimental.pallas import tpu as pltpu
from jax.experimental.pallas import tpu_sc as plsc

N = 10000
E = 320000
IN_CH, H1, H2, OUT_CH = 128, 64, 32, 1

NC, NS = 2, 16
NW = NC * NS
C = 128
CH = 80
EW = CH * C
EPAD = NW * EW
NPAD = 10112
RPT = NPAD // NS

_f32 = jnp.float32
FP = 8


def _mesh():
    return plsc.VectorSubcoreMesh(
        core_axis_name="c", subcore_axis_name="s", num_cores=NC, num_subcores=NS
    )


@functools.partial(
    pl.kernel,
    out_type=jax.ShapeDtypeStruct((NC, NPAD, FP), _f32),
    mesh=_mesh(),
    scratch_types=[
        pltpu.VMEM((CH, C), jnp.int32),
        pltpu.VMEM((C, FP), _f32),
        pltpu.VMEM_SHARED((NPAD, FP), _f32),
    ],
    compiler_params=pltpu.CompilerParams(use_tc_tiling_on_sc=False),
    name="gcn_degree",
)
def _deg_kernel(dstw, ones, zrows, out, dst_v, ones_v, acc):
    cid = lax.axis_index("c")
    sid = lax.axis_index("s")
    wid = sid * NC + cid
    r0 = sid * RPT
    pltpu.sync_copy(zrows.at[pl.ds(r0, RPT)], acc.at[pl.ds(r0, RPT)])
    pltpu.sync_copy(ones, ones_v)
    pltpu.sync_copy(dstw.at[wid], dst_v)
    plsc.subcore_barrier()

    def chunk(j, carry):
        pltpu.sync_copy(ones_v, acc.at[dst_v.at[j]], add=True)
        return carry

    lax.fori_loop(0, CH, chunk, 0)
    plsc.subcore_barrier()
    pltpu.sync_copy(acc.at[pl.ds(r0, RPT)], out.at[cid, pl.ds(r0, RPT), :])


NBUF = 4

_SPLIT = {64: (80, 80), 32: (80, 80), 8: (80, 80)}


def _make_agg(F):
    ch0, ch1 = _SPLIT[F]
    chm = max(ch0, ch1)
    stream_dst = max(ch0, ch1) > 96 and F >= 64
    dst_scr = (pltpu.VMEM((2, NBUF, C), jnp.int32) if stream_dst
               else pltpu.VMEM((chm, C), jnp.int32))

    @functools.partial(
        pl.kernel,
        out_type=jax.ShapeDtypeStruct((NC, NPAD, F), _f32),
        mesh=_mesh(),
        scratch_types=[
            pltpu.VMEM((chm, C), jnp.int32),
            dst_scr,
            pltpu.VMEM((2, NBUF, C, F), _f32),
            pltpu.VMEM_SHARED((NPAD, F), _f32),
            pltpu.SemaphoreType.DMA((2, NBUF)),
            pltpu.SemaphoreType.DMA((2, NBUF)),
            pltpu.SemaphoreType.DMA((2, NBUF)),
        ],
        compiler_params=pltpu.CompilerParams(use_tc_tiling_on_sc=False),
        name=f"gcn_agg_f{F}",
    )
    def _agg(hs, srcw, dstw, zrows, out, src_v, dst_v, buf, acc, gsem, ssem,
             dsem):
        cid = lax.axis_index("c")
        sid = lax.axis_index("s")
        wid = cid * NS + sid
        ngrp = jnp.where(cid == 0, ch0 // NBUF, ch1 // NBUF)
        r0 = sid * RPT
        pltpu.sync_copy(zrows.at[pl.ds(r0, RPT)], acc.at[pl.ds(r0, RPT)])
        pltpu.sync_copy(srcw.at[wid], src_v)
        if not stream_dst:
            pltpu.sync_copy(dstw.at[wid], dst_v)
        plsc.subcore_barrier()

        def start_gather(r, b, j):
            pltpu.async_copy(hs.at[src_v.at[j]], buf.at[r, b], gsem.at[r, b])
            if stream_dst:
                pltpu.async_copy(dstw.at[wid, j], dst_v.at[r, b],
                                 dsem.at[r, b])

        def wait_gather(r, b, j):
            pltpu.make_async_copy(hs.at[src_v.at[j]], buf.at[r, b],
                                  gsem.at[r, b]).wait()
            if stream_dst:
                pltpu.make_async_copy(dstw.at[wid, j], dst_v.at[r, b],
                                      dsem.at[r, b]).wait()

        def _dst_idx(r, b, j):
            return dst_v.at[r, b] if stream_dst else dst_v.at[j]

        def start_scatter(r, b, j):
            pltpu.async_copy(buf.at[r, b], acc.at[_dst_idx(r, b, j)],
                             ssem.at[r, b], add=True)

        def wait_scatter(r, b, j):
            pltpu.make_async_copy(buf.at[r, b], acc.at[_dst_idx(r, b, j)],
                                  ssem.at[r, b]).wait()

        for b in range(NBUF):
            start_gather(0, b, b)
        for b in range(NBUF):
            start_gather(1, b, NBUF + b)

        def pair(gg, carry):
            e0 = (2 * gg) * NBUF
            o0 = e0 + NBUF
            for b in range(NBUF):
                wait_gather(0, b, e0 + b)
                start_scatter(0, b, e0 + b)
            for b in range(NBUF):
                wait_scatter(0, b, e0 + b)
                start_gather(0, b, e0 + 2 * NBUF + b)
            for b in range(NBUF):
                wait_gather(1, b, o0 + b)
                start_scatter(1, b, o0 + b)
            for b in range(NBUF):
                wait_scatter(1, b, o0 + b)
                start_gather(1, b, o0 + 2 * NBUF + b)
            return carry

        lax.fori_loop(0, ngrp // 2 - 1, pair, 0)

        eb = (ngrp - 2) * NBUF
        ob = (ngrp - 1) * NBUF
        for b in range(NBUF):
            wait_gather(0, b, eb + b)
            start_scatter(0, b, eb + b)
        for b in range(NBUF):
            wait_gather(1, b, ob + b)
            start_scatter(1, b, ob + b)
        for b in range(NBUF):
            wait_scatter(0, b, eb + b)
        for b in range(NBUF):
            wait_scatter(1, b, ob + b)

        plsc.subcore_barrier()
        pltpu.sync_copy(acc.at[pl.ds(r0, RPT)], out.at[cid, pl.ds(r0, RPT), :])

    return _agg


_agg64 = _make_agg(H1)
_agg32 = _make_agg(H2)
_agg8 = _make_agg(FP)


BM = 512


def _tc1_body(x_ref, w_ref, cnt_ref, h_ref, hs_ref, dis_ref, dinv_ref):
    deg = cnt_ref[0, :, 0:1] + cnt_ref[1, :, 0:1] + 1.0
    dis = lax.rsqrt(deg)
    dinv = 1.0 / deg
    h = jnp.dot(x_ref[...], w_ref[...], preferred_element_type=_f32)
    h_ref[...] = h
    hs_ref[...] = dis * h
    dis_ref[...] = dis
    dinv_ref[...] = dinv


def _tc_mid_body(agg_ref, h_ref, dis_ref, dinv_ref, b_ref, w_ref, h2_ref, hs2_ref,
                 *, fout, fpad):
    dis = dis_ref[...]
    z = dis * (agg_ref[0] + agg_ref[1]) + dinv_ref[...] * h_ref[...] + b_ref[...]
    a = jnp.maximum(z, 0.0)
    h2 = jnp.dot(a, w_ref[...], preferred_element_type=_f32)
    h2_ref[...] = h2
    hs = dis * h2
    if fpad == fout:
        hs2_ref[...] = hs
    else:
        col = lax.broadcasted_iota(jnp.int32, (BM, fpad), 1)
        hs2_ref[...] = jnp.where(col < fout, hs, 0.0)


def _tc_out_body(agg_ref, h_ref, dis_ref, dinv_ref, b_ref, out_ref):
    out_ref[...] = (
        dis_ref[...] * (agg_ref[0, :, 0:1] + agg_ref[1, :, 0:1])
        + dinv_ref[...] * h_ref[...]
        + b_ref[...]
    )


def _row_spec(f):
    return pl.BlockSpec((BM, f), lambda i: (i, 0))


def _agg_spec(f):
    return pl.BlockSpec((NC, BM, f), lambda i: (0, i, 0))


def _full_spec(shape):
    return pl.BlockSpec(shape, lambda i: tuple(0 for _ in shape))


_GRID = (pl.cdiv(N, BM),)


def _tc1(x, w1, cnt):
    return pl.pallas_call(
        _tc1_body,
        grid=_GRID,
        in_specs=[_row_spec(IN_CH), _full_spec((IN_CH, H1)), _agg_spec(FP)],
        out_specs=[_row_spec(H1), _row_spec(H1), _row_spec(1), _row_spec(1)],
        out_shape=[
            jax.ShapeDtypeStruct((N, H1), _f32),
            jax.ShapeDtypeStruct((N, H1), _f32),
            jax.ShapeDtypeStruct((N, 1), _f32),
            jax.ShapeDtypeStruct((N, 1), _f32),
        ],
    )(x, w1, cnt)


def _tc_mid(agg, h, dis, dinv, b, w, fin, fout, fpad=None):
    fpad = fout if fpad is None else fpad
    return pl.pallas_call(
        functools.partial(_tc_mid_body, fout=fout, fpad=fpad),
        grid=_GRID,
        in_specs=[
            _agg_spec(fin),
            _row_spec(fin),
            _row_spec(1),
            _row_spec(1),
            _full_spec((1, fin)),
            _full_spec((fin, fout)),
        ],
        out_specs=[_row_spec(fout), _row_spec(fpad)],
        out_shape=[
            jax.ShapeDtypeStruct((N, fout), _f32),
            jax.ShapeDtypeStruct((N, fpad), _f32),
        ],
    )(agg, h, dis, dinv, b, w)


def _tc_out(agg, h, dis, dinv, b):
    return pl.pallas_call(
        _tc_out_body,
        grid=_GRID,
        in_specs=[
            _agg_spec(FP),
            _row_spec(1),
            _row_spec(1),
            _row_spec(1),
            _full_spec((1, 1)),
        ],
        out_specs=_row_spec(1),
        out_shape=jax.ShapeDtypeStruct((N, 1), _f32),
    )(agg, h, dis, dinv, b)


def kernel(x, edge_index, W1, b1, W2, b2, W3, b3):
    src = edge_index[0].astype(jnp.int32)
    dst = edge_index[1].astype(jnp.int32)
    pad = EPAD - E
    pad_iota = jnp.arange(pad, dtype=jnp.int32)
    pad_src = pad_iota % N
    pad_dst = N + (pad_iota % (NPAD - N))
    srcf = jnp.concatenate([src, pad_src])
    dstf = jnp.concatenate([dst, pad_dst])
    dstp = dstf.reshape(NW, CH, C)

    def _asym(a, fill, ch0, ch1):
        chm = max(ch0, ch1)
        cap0 = NS * ch0 * C
        a0 = a[:cap0].reshape(NS, ch0, C)
        a1 = a[cap0:].reshape(NS, ch1, C)
        pad1 = jnp.full((NS, chm - ch1, C), fill, jnp.int32)
        return jnp.concatenate([a0, jnp.concatenate([a1, pad1], axis=1)], axis=0)

    layouts = {
        f: (_asym(srcf, 0, c0, c1), _asym(dstf, N, c0, c1))
        for f, (c0, c1) in _SPLIT.items()
    }

    z64 = jnp.zeros((NPAD, H1), _f32)
    z32 = jnp.zeros((NPAD, H2), _f32)
    z8 = jnp.zeros((NPAD, FP), _f32)
    ones = jnp.ones((C, FP), _f32)

    cnt = _deg_kernel(dstp, ones, z8)
    h1, hs1, dis, dinv = _tc1(x, W1, cnt)
    agg1 = _agg64(hs1, *layouts[H1], z64)
    h2, hs2 = _tc_mid(agg1, h1, dis, dinv, b1.reshape(1, H1), W2, H1, H2)
    agg2 = _agg32(hs2, *layouts[H2], z32)
    h3, hs3 = _tc_mid(agg2, h2, dis, dinv, b2.reshape(1, H2), W3, H2, OUT_CH, FP)
    agg3 = _agg8(hs3, *layouts[FP], z8)
    return _tc_out(agg3, h3, dis, dinv, b3.reshape(1, 1))


# speedup vs baseline: 4.1286x; 1.1130x over previous
import functools

import jax
import jax.numpy as jnp
from jax import lax
from jax.experimental import pallas as pl
from jax.experimental.pallas import tpu as pltpu
from jax.experimental.pallas import tpu_sc as plsc

N = 10000
E = 320000
IN_CH, H1, H2, OUT_CH = 128, 64, 32, 1

NC, NS = 2, 16
NW = NC * NS
C = 128
CH = 80
EW = CH * C
EPAD = NW * EW
NPAD = 10112
RPT = NPAD // NS

_f32 = jnp.float32
FP = 8


def _mesh():
    return plsc.VectorSubcoreMesh(
        core_axis_name="c", subcore_axis_name="s", num_cores=NC, num_subcores=NS
    )


@functools.partial(
    pl.kernel,
    out_type=jax.ShapeDtypeStruct((NC, NPAD, FP), _f32),
    mesh=_mesh(),
    scratch_types=[
        pltpu.VMEM((CH, C), jnp.int32),
        pltpu.VMEM((C, FP), _f32),
        pltpu.VMEM_SHARED((NPAD, FP), _f32),
    ],
    compiler_params=pltpu.CompilerParams(use_tc_tiling_on_sc=False),
    name="gcn_degree",
)
def _deg_kernel(dstw, ones, zrows, out, dst_v, ones_v, acc):
    cid = lax.axis_index("c")
    sid = lax.axis_index("s")
    wid = cid * NS + sid
    r0 = sid * RPT
    pltpu.sync_copy(zrows.at[pl.ds(r0, RPT)], acc.at[pl.ds(r0, RPT)])
    pltpu.sync_copy(ones, ones_v)
    pltpu.sync_copy(dstw.at[wid], dst_v)
    plsc.subcore_barrier()

    def chunk(j, carry):
        pltpu.sync_copy(ones_v, acc.at[dst_v.at[j]], add=True)
        return carry

    lax.fori_loop(0, CH, chunk, 0)
    plsc.subcore_barrier()
    pltpu.sync_copy(acc.at[pl.ds(r0, RPT)], out.at[cid, pl.ds(r0, RPT), :])


NBUF = 4

_SPLIT = {64: (80, 80), 32: (80, 80), 8: (80, 80)}


def _make_agg(F):
    ch0, ch1 = _SPLIT[F]
    chm = max(ch0, ch1)
    stream_dst = max(ch0, ch1) > 96 and F >= 64
    dst_scr = (pltpu.VMEM((2, NBUF, C), jnp.int32) if stream_dst
               else pltpu.VMEM((chm, C), jnp.int32))

    @functools.partial(
        pl.kernel,
        out_type=jax.ShapeDtypeStruct((NC, NPAD, F), _f32),
        mesh=_mesh(),
        scratch_types=[
            pltpu.VMEM((chm, C), jnp.int32),
            dst_scr,
            pltpu.VMEM((2, NBUF, C, F), _f32),
            pltpu.VMEM_SHARED((NPAD, F), _f32),
            pltpu.SemaphoreType.DMA((2, NBUF)),
            pltpu.SemaphoreType.DMA((2, NBUF)),
            pltpu.SemaphoreType.DMA((2, NBUF)),
        ],
        compiler_params=pltpu.CompilerParams(use_tc_tiling_on_sc=False),
        name=f"gcn_agg_f{F}",
    )
    def _agg(hs, srcw, dstw, zrows, out, src_v, dst_v, buf, acc, gsem, ssem,
             dsem):
        cid = lax.axis_index("c")
        sid = lax.axis_index("s")
        wid = cid * NS + sid
        ngrp = jnp.where(cid == 0, ch0 // NBUF, ch1 // NBUF)
        r0 = sid * RPT
        pltpu.sync_copy(zrows.at[pl.ds(r0, RPT)], acc.at[pl.ds(r0, RPT)])
        pltpu.sync_copy(srcw.at[wid], src_v)
        if not stream_dst:
            pltpu.sync_copy(dstw.at[wid], dst_v)
        plsc.subcore_barrier()

        def start_gather(r, b, j):
            pltpu.async_copy(hs.at[src_v.at[j]], buf.at[r, b], gsem.at[r, b])
            if stream_dst:
                pltpu.async_copy(dstw.at[wid, j], dst_v.at[r, b],
                                 dsem.at[r, b])

        def wait_gather(r, b, j):
            pltpu.make_async_copy(hs.at[src_v.at[j]], buf.at[r, b],
                                  gsem.at[r, b]).wait()
            if stream_dst:
                pltpu.make_async_copy(dstw.at[wid, j], dst_v.at[r, b],
                                      dsem.at[r, b]).wait()

        def _dst_idx(r, b, j):
            return dst_v.at[r, b] if stream_dst else dst_v.at[j]

        def start_scatter(r, b, j):
            pltpu.async_copy(buf.at[r, b], acc.at[_dst_idx(r, b, j)],
                             ssem.at[r, b], add=True)

        def wait_scatter(r, b, j):
            pltpu.make_async_copy(buf.at[r, b], acc.at[_dst_idx(r, b, j)],
                                  ssem.at[r, b]).wait()

        for b in range(NBUF):
            start_gather(0, b, b)
        for b in range(NBUF):
            start_gather(1, b, NBUF + b)

        def pair(gg, carry):
            e0 = (2 * gg) * NBUF
            o0 = e0 + NBUF
            for b in range(NBUF):
                wait_gather(0, b, e0 + b)
                start_scatter(0, b, e0 + b)
            for b in range(NBUF):
                wait_scatter(0, b, e0 + b)
                start_gather(0, b, e0 + 2 * NBUF + b)
            for b in range(NBUF):
                wait_gather(1, b, o0 + b)
                start_scatter(1, b, o0 + b)
            for b in range(NBUF):
                wait_scatter(1, b, o0 + b)
                start_gather(1, b, o0 + 2 * NBUF + b)
            return carry

        lax.fori_loop(0, ngrp // 2 - 1, pair, 0)

        eb = (ngrp - 2) * NBUF
        ob = (ngrp - 1) * NBUF
        for b in range(NBUF):
            wait_gather(0, b, eb + b)
            start_scatter(0, b, eb + b)
        for b in range(NBUF):
            wait_gather(1, b, ob + b)
            start_scatter(1, b, ob + b)
        for b in range(NBUF):
            wait_scatter(0, b, eb + b)
        for b in range(NBUF):
            wait_scatter(1, b, ob + b)

        plsc.subcore_barrier()
        pltpu.sync_copy(acc.at[pl.ds(r0, RPT)], out.at[cid, pl.ds(r0, RPT), :])

    return _agg


_agg64 = _make_agg(H1)
_agg32 = _make_agg(H2)
_agg8 = _make_agg(FP)


BM = 1024


def _tc1_body(x_ref, w_ref, cnt_ref, hs_ref, dis_ref):
    deg = cnt_ref[0, :, 0:1] + cnt_ref[1, :, 0:1] + 1.0
    dis = lax.rsqrt(deg)
    h = jnp.dot(x_ref[...], w_ref[...], preferred_element_type=_f32)
    hs_ref[...] = dis * h
    dis_ref[...] = dis


def _tc_mid_body(agg_ref, hp_ref, dis_ref, b_ref, w_ref, hs2_ref,
                 *, fin, fout, fpad):
    dis = dis_ref[...]
    del fin
    z = dis * (agg_ref[0] + agg_ref[1] + hp_ref[...]) + b_ref[...]
    a = jnp.maximum(z, 0.0)
    h2 = jnp.dot(a, w_ref[...], preferred_element_type=_f32)
    hs = dis * h2
    if fpad == fout:
        hs2_ref[...] = hs
    else:
        col = lax.broadcasted_iota(jnp.int32, (BM, fpad), 1)
        hs2_ref[...] = jnp.where(col < fout, hs, 0.0)


def _tc_out_body(agg_ref, hp_ref, dis_ref, b_ref, out_ref):
    out_ref[...] = (
        dis_ref[...]
        * (agg_ref[0, :, 0:1] + agg_ref[1, :, 0:1] + hp_ref[:, 0:1])
        + b_ref[...]
    )


def _row_spec(f):
    return pl.BlockSpec((BM, f), lambda i: (i, 0))


def _agg_spec(f):
    return pl.BlockSpec((NC, BM, f), lambda i: (0, i, 0))


def _full_spec(shape):
    return pl.BlockSpec(shape, lambda i: tuple(0 for _ in shape))


_GRID = (pl.cdiv(N, BM),)


def _tc1(x, w1, cnt):
    return pl.pallas_call(
        _tc1_body,
        grid=_GRID,
        in_specs=[_row_spec(IN_CH), _full_spec((IN_CH, H1)), _agg_spec(FP)],
        out_specs=[_row_spec(H1), _row_spec(1)],
        out_shape=[
            jax.ShapeDtypeStruct((N, H1), _f32),
            jax.ShapeDtypeStruct((N, 1), _f32),
        ],
    )(x, w1, cnt)


def _tc_mid(agg, hp, dis, b, w, fin, fout, fpad=None):
    fpad = fout if fpad is None else fpad
    return pl.pallas_call(
        functools.partial(_tc_mid_body, fin=fin, fout=fout, fpad=fpad),
        grid=_GRID,
        in_specs=[
            _agg_spec(fin),
            _row_spec(fin),
            _row_spec(1),
            _full_spec((1, fin)),
            _full_spec((fin, fout)),
        ],
        out_specs=_row_spec(fpad),
        out_shape=jax.ShapeDtypeStruct((N, fpad), _f32),
    )(agg, hp, dis, b, w)


def _tc_out(agg, hp, dis, b):
    return pl.pallas_call(
        _tc_out_body,
        grid=_GRID,
        in_specs=[
            _agg_spec(FP),
            _row_spec(FP),
            _row_spec(1),
            _full_spec((1, 1)),
        ],
        out_specs=_row_spec(1),
        out_shape=jax.ShapeDtypeStruct((N, 1), _f32),
    )(agg, hp, dis, b)


def kernel(x, edge_index, W1, b1, W2, b2, W3, b3):
    src = edge_index[0].astype(jnp.int32)
    dst = edge_index[1].astype(jnp.int32)
    pad = EPAD - E
    pad_iota = jnp.arange(pad, dtype=jnp.int32)
    pad_src = pad_iota % N
    pad_dst = N + (pad_iota % (NPAD - N))
    srcf = jnp.concatenate([src, pad_src])
    dstf = jnp.concatenate([dst, pad_dst])
    dstp = dstf.reshape(NW, CH, C)

    srcp = srcf.reshape(NW, CH, C)

    z64 = jnp.zeros((NPAD, H1), _f32)
    z32 = jnp.zeros((NPAD, H2), _f32)
    z8 = jnp.zeros((NPAD, FP), _f32)
    ones = jnp.ones((C, FP), _f32)

    cnt = _deg_kernel(dstp, ones, z8)
    hs1, dis = _tc1(x, W1, cnt)
    agg1 = _agg64(hs1, srcp, dstp, z64)
    hs2 = _tc_mid(agg1, hs1, dis, b1.reshape(1, H1), W2, H1, H2)
    agg2 = _agg32(hs2, srcp, dstp, z32)
    hs3 = _tc_mid(agg2, hs2, dis, b2.reshape(1, H2), W3, H2, OUT_CH, FP)
    agg3 = _agg8(hs3, srcp, dstp, z8)
    return _tc_out(agg3, hs3, dis, b3.reshape(1, 1))
